# Initial kernel scaffold; baseline (speedup 1.0000x reference)
import jax
import jax.numpy as jnp
from jax.experimental import pallas as pl


def kernel(x, conv_w, conv_b, lin_w, lin_b, rowkeys, colkeys):
    raise NotImplementedError("write your pallas kernel here")


# speedup vs baseline: 5.4842x; 5.4842x over previous
import jax
import jax.numpy as jnp
from jax import lax
from jax.experimental import pallas as pl
from jax.experimental.pallas import tpu as pltpu

HEADS = 4
KD2 = 256
BLOCK = 64
COL = 1024
KNN = 32

TAU_BLK = 256

_INTERPRET = False


def _score_body(x_ref, xprev_ref, cw_ref, cb_ref, lw_ref, lb_ref, rk_ref, ck_ref,
                s_ref, mc_ref):
    t = pl.program_id(1)
    x = x_ref[0]
    prev = xprev_ref[0]
    first = t == 0
    prev2 = jnp.where(first, 0.0, prev[6:8])
    xm1 = jnp.concatenate([prev2[1:2], x[:-1]], axis=0)
    xm2 = jnp.concatenate([prev2[0:2], x[:-2]], axis=0)
    xc = xm2 * cw_ref[0:1] + xm1 * cw_ref[1:2] + x * cw_ref[2:3] + cb_ref[0:1]
    q = lax.dot_general(xc, lw_ref[...], (((1,), (1,)), ((), ())),
                        preferred_element_type=jnp.float32)
    q = q + lb_ref[0:1]

    for r in range(4):
        qr = q[:, r * KD2:(r + 1) * KD2]
        rs = lax.dot_general(qr, rk_ref[0], (((1,), (1,)), ((), ())),
                             preferred_element_type=jnp.float32)
        cs = lax.dot_general(qr, ck_ref[0], (((1,), (1,)), ((), ())),
                             preferred_element_type=jnp.float32)
        ms = jnp.max(cs, axis=1, keepdims=True)
        iot = lax.broadcasted_iota(jnp.int32, cs.shape, 1)
        mc = jnp.min(jnp.where(cs == ms, iot, COL), axis=1, keepdims=True)
        s_ref[0, :, r * BLOCK:(r + 1) * BLOCK] = rs + ms
        iot_e = lax.broadcasted_iota(jnp.int32, (TAU_BLK, BLOCK), 1)
        tau_loc = lax.broadcasted_iota(jnp.int32, (TAU_BLK, BLOCK), 0)
        jv = (4 * (tau_loc % 8) + r) * BLOCK + iot_e
        ind = iot_e * COL + mc
        mc_ref[0, :, r * BLOCK:(r + 1) * BLOCK] = ind * 2048 + jv


def _scores(x, conv_w, conv_b, lin_w, lin_b, rowkeys, colkeys):
    B, T, C = x.shape
    nt = T // TAU_BLK
    cw = conv_w.T
    cb = conv_b.reshape(1, C)
    lb = lin_b.reshape(1, C)
    rk = rowkeys.transpose(1, 0, 2)
    ck = colkeys.transpose(1, 0, 2)
    grid = (B, nt)
    s, mc = pl.pallas_call(
        _score_body,
        grid=grid,
        in_specs=[
            pl.BlockSpec((1, TAU_BLK, C), lambda b, t: (b, t, 0)),
            pl.BlockSpec((1, 8, C), lambda b, t: (b, jnp.maximum(t * (TAU_BLK // 8) - 1, 0), 0)),
            pl.BlockSpec((3, C), lambda b, t: (0, 0)),
            pl.BlockSpec((1, C), lambda b, t: (0, 0)),
            pl.BlockSpec((C, C), lambda b, t: (0, 0)),
            pl.BlockSpec((1, C), lambda b, t: (0, 0)),
            pl.BlockSpec((1, BLOCK, KD2), lambda b, t: (t // 2, 0, 0)),
            pl.BlockSpec((1, COL, KD2), lambda b, t: (t // 2, 0, 0)),
        ],
        out_specs=[
            pl.BlockSpec((1, TAU_BLK, 4 * BLOCK), lambda b, t: (b, t, 0)),
            pl.BlockSpec((1, TAU_BLK, 4 * BLOCK), lambda b, t: (b, t, 0)),
        ],
        out_shape=[
            jax.ShapeDtypeStruct((B, T, 4 * BLOCK), jnp.float32),
            jax.ShapeDtypeStruct((B, T, 4 * BLOCK), jnp.int32),
        ],
        interpret=_INTERPRET,
    )(x, x, cw, cb, lin_w, lb, rk, ck)
    return s, mc


def _sort_body(key_ref, pay_ref, sg_ref, ig_ref, dp_ref):
    kv = key_ref[0]
    pv = pay_ref[0]
    pos = lax.broadcasted_iota(jnp.int32, kv.shape, 1)
    kk = 2
    while kk <= 2048:
        jj = kk // 2
        while jj >= 1:
            upper = (pos & jj) != 0
            dirdesc = (pos & kk) == 0
            pk = jnp.where(upper, jnp.roll(kv, jj, 1), jnp.roll(kv, -jj, 1))
            pp = jnp.where(upper, jnp.roll(pv, jj, 1), jnp.roll(pv, -jj, 1))
            jself = pv & 2047
            jpart = pp & 2047
            better = (kv > pk) | ((kv == pk) & (jself < jpart))
            wantsb = (~upper) == dirdesc
            take_self = better == wantsb
            kv = jnp.where(take_self, kv, pk)
            pv = jnp.where(take_self, pv, pp)
            jj //= 2
        kk *= 2
    sg_ref[0] = kv[:, :1024]
    dp_ref[0] = pv[:, :1024] & 2047
    ig_ref[0] = lax.shift_right_logical(pv[:, :1024], 11)


def _sort_rows(key2, pay2):
    R = key2.shape[0]
    return pl.pallas_call(
        _sort_body,
        grid=(R,),
        in_specs=[
            pl.BlockSpec((1, BLOCK, 2048), lambda g: (g, 0, 0)),
            pl.BlockSpec((1, BLOCK, 2048), lambda g: (g, 0, 0)),
        ],
        out_specs=[
            pl.BlockSpec((1, BLOCK, 1024), lambda g: (g, 0, 0)),
            pl.BlockSpec((1, BLOCK, 1024), lambda g: (g, 0, 0)),
            pl.BlockSpec((1, BLOCK, 1024), lambda g: (g, 0, 0)),
        ],
        out_shape=[
            jax.ShapeDtypeStruct((R, BLOCK, 1024), jnp.float32),
            jax.ShapeDtypeStruct((R, BLOCK, 1024), jnp.int32),
            jax.ShapeDtypeStruct((R, BLOCK, 1024), jnp.int32),
        ],
        interpret=_INTERPRET,
    )(key2, pay2)


def kernel(x, conv_w, conv_b, lin_w, lin_b, rowkeys, colkeys):
    B, T, C = x.shape
    k = T * KNN // BLOCK
    s, pay = _scores(x, conv_w, conv_b, lin_w, lin_b, rowkeys, colkeys)
    key2 = s.reshape(B * HEADS, BLOCK, T)
    pay2 = pay.reshape(B * HEADS, BLOCK, T)
    score_g, indices_g, dispatch = _sort_rows(key2, pay2)
    return (score_g.reshape(B, BLOCK, -1), indices_g.reshape(B, BLOCK, -1),
            dispatch.reshape(B, BLOCK, -1))
